# baseline (speedup 1.0000x reference)
import functools

import jax
import jax.numpy as jnp
from jax import lax
from jax.experimental import pallas as pl
from jax.experimental.pallas import tpu as pltpu
from jax.experimental.pallas import tpu_sc as plsc

NUM_EMB = 1_000_000
DIM = 64
B_TOTAL = 16384 * 20
LANE = 128
N_WORKERS = 32
ROWS_TOTAL = B_TOTAL // LANE
ROWS_PER_W = ROWS_TOTAL // N_WORKERS
KCH = 4
N_CHUNKS = ROWS_PER_W // KCH


UCW = 128
UNCH = NUM_EMB // UCW
UCH_PER_W = UNCH // N_WORKERS
UNCH_REM = UNCH - UCH_PER_W * N_WORKERS
N_PAIR = NUM_EMB // 2


def _sc_untile(table):
    mesh = plsc.VectorSubcoreMesh(core_axis_name="c", subcore_axis_name="s")

    @functools.partial(
        pl.kernel,
        out_type=jax.ShapeDtypeStruct((N_PAIR, 128), jnp.float32),
        mesh=mesh,
        scratch_types=[
            pltpu.VMEM((UCW, 64), jnp.float32),
            pltpu.VMEM((UCW, 64), jnp.float32),
            pltpu.VMEM((UCW // 2, 128), jnp.float32),
            pltpu.VMEM((UCW // 2, 128), jnp.float32),
            pltpu.SemaphoreType.DMA,
            pltpu.SemaphoreType.DMA,
            pltpu.SemaphoreType.DMA,
            pltpu.SemaphoreType.DMA,
        ],
        compiler_params=pltpu.CompilerParams(
            use_tc_tiling_on_sc=True, needs_layout_passes=False
        ),
    )
    def k(tab_hbm, lin_hbm, a0, a1, b0, b1, si0, si1, so0, so1):
        wid = lax.axis_index("s") * 2 + lax.axis_index("c")
        ch0 = wid * UCH_PER_W
        abuf = (a0, a1)
        bbuf = (b0, b1)
        sis = (si0, si1)
        sos = (so0, so1)

        def in_copy(g, b):
            r0 = pl.multiple_of((ch0 + g) * UCW, UCW)
            return pltpu.make_async_copy(
                tab_hbm.at[pl.ds(r0, UCW), :], abuf[b], sis[b]
            )

        def out_copy(g, b):
            k0 = pl.multiple_of((ch0 + g) * (UCW // 2), UCW // 2)
            return pltpu.make_async_copy(
                bbuf[b], lin_hbm.at[pl.ds(k0, UCW // 2)], sos[b]
            )

        def repack(b, npair):
            @plsc.parallel_loop(0, npair, step=1, unroll=16)
            def _(j):
                for half in range(2):
                    for cb in range(0, 64, 16):
                        v = abuf[b][2 * j + half, pl.ds(cb, 16)]
                        bbuf[b][j, pl.ds(half * 64 + cb, 16)] = v

        in_copy(0, 0).start()

        def outer(gg, carry):
            for b in range(2):
                g = gg * 2 + b

                @pl.when(g + 1 < UCH_PER_W)
                def _():
                    in_copy(g + 1, 1 - b).start()

                in_copy(g, b).wait()

                @pl.when(g >= 2)
                def _():
                    out_copy(g - 2, b).wait()

                repack(b, UCW // 2)
                out_copy(g, b).start()
            return carry

        lax.fori_loop(0, UCH_PER_W // 2, outer, 0)
        out_copy(UCH_PER_W - 2, 0).wait()
        out_copy(UCH_PER_W - 1, 1).wait()

        @pl.when(wid < UNCH_REM)
        def _():
            g = UCH_PER_W * N_WORKERS + wid - ch0
            in_copy(g, 0).start()
            in_copy(g, 0).wait()
            repack(0, UCW // 2)
            out_copy(g, 0).start()
            out_copy(g, 0).wait()

        @pl.when(wid == UNCH_REM)
        def _():
            pltpu.sync_copy(
                tab_hbm.at[pl.ds(NUM_EMB - 64, 64), :], a1.at[pl.ds(0, 64)]
            )
            repack(1, 32)
            pltpu.sync_copy(
                b1.at[pl.ds(0, 32)], lin_hbm.at[pl.ds(N_PAIR - 32, 32)]
            )

    return k(table)


def _sc_gather(x2, table):
    mesh = plsc.VectorSubcoreMesh(core_axis_name="c", subcore_axis_name="s")

    @functools.partial(
        pl.kernel,
        out_type=jax.ShapeDtypeStruct((ROWS_TOTAL, LANE, DIM), jnp.float32),
        mesh=mesh,
        scratch_types=[
            pltpu.VMEM((KCH, LANE), jnp.int32),
            pltpu.VMEM((KCH, LANE), jnp.int32),
            pltpu.VMEM((KCH, LANE, DIM), jnp.float32),
            pltpu.VMEM((KCH, LANE, DIM), jnp.float32),
            pltpu.SemaphoreType.DMA,
            pltpu.SemaphoreType.DMA,
        ],
        compiler_params=pltpu.CompilerParams(use_tc_tiling_on_sc=False),
    )
    def k(x_hbm, tab_hbm, out_hbm, idx0, idx1, rows0, rows1, sem0, sem1):
        wid = lax.axis_index("s") * 2 + lax.axis_index("c")
        row0 = wid * ROWS_PER_W
        idx_b = (idx0, idx1)
        rows_b = (rows0, rows1)
        sem_b = (sem0, sem1)

        def fire(g, b):
            r = row0 + g * KCH
            pltpu.sync_copy(x_hbm.at[pl.ds(r, KCH)], idx_b[b])
            for j in range(KCH):
                pltpu.async_copy(
                    tab_hbm.at[idx_b[b].at[j]], rows_b[b].at[j], sem_b[b]
                )

        def drain_store(g, b):
            for j in range(KCH):
                pltpu.make_async_copy(
                    tab_hbm.at[idx_b[b].at[j]], rows_b[b].at[j], sem_b[b]
                ).wait()
            pltpu.sync_copy(rows_b[b], out_hbm.at[pl.ds(row0 + g * KCH, KCH)])

        fire(0, 0)

        def outer(gg, carry):
            for b in range(2):
                g = gg * 2 + b

                @pl.when(g + 1 < N_CHUNKS)
                def _():
                    fire(g + 1, 1 - b)

                drain_store(g, b)
            return carry

        lax.fori_loop(0, N_CHUNKS // 2, outer, 0)

    return k(x2, table)


def kernel(x, A):
    x2 = x.T.reshape(ROWS_TOTAL, LANE).astype(jnp.int32)
    lin = _sc_untile(A).reshape(NUM_EMB, DIM)
    out = _sc_gather(x2, lin)
    return out.reshape(20, 16384, DIM).transpose(1, 0, 2)


# speedup vs baseline: 1.2871x; 1.2871x over previous
import functools

import jax
import jax.numpy as jnp
from jax import lax
from jax.experimental import pallas as pl
from jax.experimental.pallas import tpu as pltpu
from jax.experimental.pallas import tpu_sc as plsc

NUM_EMB = 1_000_000
DIM = 64
B_TOTAL = 16384 * 20
LANE = 128
N_WORKERS = 32
ROWS_TOTAL = B_TOTAL // LANE
ROWS_PER_W = ROWS_TOTAL // N_WORKERS
KCH = 4
N_CHUNKS = ROWS_PER_W // KCH


TCB = 4096
TGRID = -(-NUM_EMB // TCB)
N_PAIR = NUM_EMB // 2


def _tc_pack(at):
    def body(in_ref, out_ref):
        z = in_ref[...].T.reshape(TCB // 2, 2, DIM)
        out_ref[...] = jnp.concatenate([z[:, 0, :], z[:, 1, :]], axis=1)

    return pl.pallas_call(
        body,
        grid=(TGRID,),
        in_specs=[pl.BlockSpec((DIM, TCB), lambda i: (0, i))],
        out_specs=pl.BlockSpec((TCB // 2, 128), lambda i: (i, 0)),
        out_shape=jax.ShapeDtypeStruct((N_PAIR, 128), jnp.float32),
    )(at)


def _sc_gather(x2, table):
    mesh = plsc.VectorSubcoreMesh(core_axis_name="c", subcore_axis_name="s")

    @functools.partial(
        pl.kernel,
        out_type=jax.ShapeDtypeStruct((ROWS_TOTAL, LANE, DIM), jnp.float32),
        mesh=mesh,
        scratch_types=[
            pltpu.VMEM((KCH, LANE), jnp.int32),
            pltpu.VMEM((KCH, LANE), jnp.int32),
            pltpu.VMEM((KCH, LANE, DIM), jnp.float32),
            pltpu.VMEM((KCH, LANE, DIM), jnp.float32),
            pltpu.SemaphoreType.DMA,
            pltpu.SemaphoreType.DMA,
        ],
        compiler_params=pltpu.CompilerParams(use_tc_tiling_on_sc=False),
    )
    def k(x_hbm, tab_hbm, out_hbm, idx0, idx1, rows0, rows1, sem0, sem1):
        wid = lax.axis_index("s") * 2 + lax.axis_index("c")
        row0 = wid * ROWS_PER_W
        idx_b = (idx0, idx1)
        rows_b = (rows0, rows1)
        sem_b = (sem0, sem1)

        def fire(g, b):
            r = row0 + g * KCH
            pltpu.sync_copy(x_hbm.at[pl.ds(r, KCH)], idx_b[b])
            for j in range(KCH):
                pltpu.async_copy(
                    tab_hbm.at[idx_b[b].at[j]], rows_b[b].at[j], sem_b[b]
                )

        def drain_store(g, b):
            for j in range(KCH):
                pltpu.make_async_copy(
                    tab_hbm.at[idx_b[b].at[j]], rows_b[b].at[j], sem_b[b]
                ).wait()
            pltpu.sync_copy(rows_b[b], out_hbm.at[pl.ds(row0 + g * KCH, KCH)])

        fire(0, 0)

        def outer(gg, carry):
            for b in range(2):
                g = gg * 2 + b

                @pl.when(g + 1 < N_CHUNKS)
                def _():
                    fire(g + 1, 1 - b)

                drain_store(g, b)
            return carry

        lax.fori_loop(0, N_CHUNKS // 2, outer, 0)

    return k(x2, table)


def kernel(x, A):
    x2 = x.T.reshape(ROWS_TOTAL, LANE).astype(jnp.int32)
    lin = _tc_pack(A.T).reshape(NUM_EMB, DIM)
    out = _sc_gather(x2, lin)
    return out.reshape(20, 16384, DIM).transpose(1, 0, 2)


# speedup vs baseline: 1.3245x; 1.0291x over previous
import functools

import jax
import jax.numpy as jnp
from jax import lax
from jax.experimental import pallas as pl
from jax.experimental.pallas import tpu as pltpu
from jax.experimental.pallas import tpu_sc as plsc

NUM_EMB = 1_000_000
DIM = 64
B_TOTAL = 16384 * 20
LANE = 128
N_WORKERS = 32
ROWS_TOTAL = B_TOTAL // LANE
ROWS_PER_W = ROWS_TOTAL // N_WORKERS
KCH = 4
N_CHUNKS = ROWS_PER_W // KCH


TCB = 8192
TGRID = -(-NUM_EMB // TCB)
N_PAIR = NUM_EMB // 2


def _tc_pack(at):
    def body(in_ref, out_ref):
        z = in_ref[...].T.reshape(TCB // 2, 2, DIM)
        out_ref[...] = jnp.concatenate([z[:, 0, :], z[:, 1, :]], axis=1)

    return pl.pallas_call(
        body,
        grid=(TGRID,),
        in_specs=[pl.BlockSpec((DIM, TCB), lambda i: (0, i))],
        out_specs=pl.BlockSpec((TCB // 2, 128), lambda i: (i, 0)),
        out_shape=jax.ShapeDtypeStruct((N_PAIR, 128), jnp.float32),
    )(at)


def _sc_gather(x2, table):
    mesh = plsc.VectorSubcoreMesh(core_axis_name="c", subcore_axis_name="s")

    @functools.partial(
        pl.kernel,
        out_type=jax.ShapeDtypeStruct((ROWS_TOTAL, LANE, DIM), jnp.float32),
        mesh=mesh,
        scratch_types=[
            pltpu.VMEM((KCH, LANE), jnp.int32),
            pltpu.VMEM((KCH, LANE), jnp.int32),
            pltpu.VMEM((KCH, LANE, DIM), jnp.float32),
            pltpu.VMEM((KCH, LANE, DIM), jnp.float32),
            pltpu.SemaphoreType.DMA,
            pltpu.SemaphoreType.DMA,
        ],
        compiler_params=pltpu.CompilerParams(use_tc_tiling_on_sc=False),
    )
    def k(x_hbm, tab_hbm, out_hbm, idx0, idx1, rows0, rows1, sem0, sem1):
        wid = lax.axis_index("s") * 2 + lax.axis_index("c")
        row0 = wid * ROWS_PER_W
        idx_b = (idx0, idx1)
        rows_b = (rows0, rows1)
        sem_b = (sem0, sem1)

        def fire(g, b):
            r = row0 + g * KCH
            pltpu.sync_copy(x_hbm.at[pl.ds(r, KCH)], idx_b[b])
            for j in range(KCH):
                pltpu.async_copy(
                    tab_hbm.at[idx_b[b].at[j]], rows_b[b].at[j], sem_b[b]
                )

        def drain_store(g, b):
            for j in range(KCH):
                pltpu.make_async_copy(
                    tab_hbm.at[idx_b[b].at[j]], rows_b[b].at[j], sem_b[b]
                ).wait()
            pltpu.sync_copy(rows_b[b], out_hbm.at[pl.ds(row0 + g * KCH, KCH)])

        fire(0, 0)

        def outer(gg, carry):
            for b in range(2):
                g = gg * 2 + b

                @pl.when(g + 1 < N_CHUNKS)
                def _():
                    fire(g + 1, 1 - b)

                drain_store(g, b)
            return carry

        lax.fori_loop(0, N_CHUNKS // 2, outer, 0)

    return k(x2, table)


def kernel(x, A):
    x2 = x.T.reshape(ROWS_TOTAL, LANE).astype(jnp.int32)
    lin = _tc_pack(A.T).reshape(NUM_EMB, DIM)
    out = _sc_gather(x2, lin)
    return out.reshape(20, 16384, DIM).transpose(1, 0, 2)


# speedup vs baseline: 1.3327x; 1.0062x over previous
import functools

import jax
import jax.numpy as jnp
from jax import lax
from jax.experimental import pallas as pl
from jax.experimental.pallas import tpu as pltpu
from jax.experimental.pallas import tpu_sc as plsc

NUM_EMB = 1_000_000
DIM = 64
B_TOTAL = 16384 * 20
LANE = 128
N_WORKERS = 32
ROWS_TOTAL = B_TOTAL // LANE
ROWS_PER_W = ROWS_TOTAL // N_WORKERS
KCH = 4
N_CHUNKS = ROWS_PER_W // KCH


TCB = 10240
TGRID = -(-NUM_EMB // TCB)
N_PAIR = NUM_EMB // 2


def _tc_pack(at):
    def body(in_ref, out_ref):
        z = in_ref[...].T.reshape(TCB // 2, 2, DIM)
        out_ref[...] = jnp.concatenate([z[:, 0, :], z[:, 1, :]], axis=1)

    return pl.pallas_call(
        body,
        grid=(TGRID,),
        in_specs=[pl.BlockSpec((DIM, TCB), lambda i: (0, i))],
        out_specs=pl.BlockSpec((TCB // 2, 128), lambda i: (i, 0)),
        out_shape=jax.ShapeDtypeStruct((N_PAIR, 128), jnp.float32),
    )(at)


def _sc_gather(x2, table):
    mesh = plsc.VectorSubcoreMesh(core_axis_name="c", subcore_axis_name="s")

    @functools.partial(
        pl.kernel,
        out_type=jax.ShapeDtypeStruct((ROWS_TOTAL, LANE, DIM), jnp.float32),
        mesh=mesh,
        scratch_types=[
            pltpu.VMEM((KCH, LANE), jnp.int32),
            pltpu.VMEM((KCH, LANE), jnp.int32),
            pltpu.VMEM((KCH, LANE, DIM), jnp.float32),
            pltpu.VMEM((KCH, LANE, DIM), jnp.float32),
            pltpu.SemaphoreType.DMA,
            pltpu.SemaphoreType.DMA,
        ],
        compiler_params=pltpu.CompilerParams(use_tc_tiling_on_sc=False),
    )
    def k(x_hbm, tab_hbm, out_hbm, idx0, idx1, rows0, rows1, sem0, sem1):
        wid = lax.axis_index("s") * 2 + lax.axis_index("c")
        row0 = wid * ROWS_PER_W
        idx_b = (idx0, idx1)
        rows_b = (rows0, rows1)
        sem_b = (sem0, sem1)

        def fire(g, b):
            r = row0 + g * KCH
            pltpu.sync_copy(x_hbm.at[pl.ds(r, KCH)], idx_b[b])
            for j in range(KCH):
                pltpu.async_copy(
                    tab_hbm.at[idx_b[b].at[j]], rows_b[b].at[j], sem_b[b]
                )

        def drain_store(g, b):
            for j in range(KCH):
                pltpu.make_async_copy(
                    tab_hbm.at[idx_b[b].at[j]], rows_b[b].at[j], sem_b[b]
                ).wait()
            pltpu.sync_copy(rows_b[b], out_hbm.at[pl.ds(row0 + g * KCH, KCH)])

        fire(0, 0)

        def outer(gg, carry):
            for b in range(2):
                g = gg * 2 + b

                @pl.when(g + 1 < N_CHUNKS)
                def _():
                    fire(g + 1, 1 - b)

                drain_store(g, b)
            return carry

        lax.fori_loop(0, N_CHUNKS // 2, outer, 0)

    return k(x2, table)


def kernel(x, A):
    x2 = x.T.reshape(ROWS_TOTAL, LANE).astype(jnp.int32)
    lin = _tc_pack(A.T).reshape(NUM_EMB, DIM)
    out = _sc_gather(x2, lin)
    return out.reshape(20, 16384, DIM).transpose(1, 0, 2)
